# baseline (speedup 1.0000x reference)
import functools

import jax
import jax.numpy as jnp
from jax import lax
from jax.experimental import pallas as pl
from jax.experimental.pallas import tpu as pltpu
from jax.experimental.pallas import tpu_sc as plsc

_K = 64
_IMIN = -(2 ** 31)
_CAP = 512
_NB1 = 2048
_NBC = 128
_NB2 = 128
_T = 4096
_NGRP = 256
_GRP_PER_W = 8


def _shr(x, n):
    return lax.shift_right_logical(x, lax.full_like(x, n))


def _skey(x):
    b = lax.bitcast_convert_type(x, jnp.int32)
    m = lax.shift_right_arithmetic(b, lax.full_like(b, 31))
    return jnp.bitwise_xor(b, lax.shift_right_logical(m, lax.full_like(m, 1)))


def _sc_body(x_hbm, o_hbm, xblk, hist, hist2, buf, obuf_i, obuf_f):
    wid = lax.axis_index("s") * 2 + lax.axis_index("c")
    lanes = lax.iota(jnp.int32, 16)
    imin = jnp.full((16,), _IMIN, jnp.int32)
    sh21 = jnp.full((16,), 21, jnp.int32)
    one16 = jnp.ones((16,), jnp.int32)
    zero16 = jnp.zeros((16,), jnp.int32)
    kvec = jnp.full((16,), _K, jnp.int32)

    def group_body(gi, _carry):
        g = wid * _GRP_PER_W + gi
        b = g // 64
        dg = g % 64
        pltpu.sync_copy(x_hbm.at[b, :, pl.ds(dg * 16, 16)], xblk)

        ns = jax.named_scope
        with ns("zero_hist"):
         def zh(i, _):
            for c in range(8):
                hist[i * 8 + c] = zero16
            return 0
         lax.fori_loop(0, _NB1 // 8, zh, 0, unroll=2)

        with ns("passA"):
         def pa(i, st):
            smax = st
            for c in range(4):
                s = _skey(xblk[i * 4 + c])
                b1 = lax.shift_right_arithmetic(s, sh21) + 1024
                plsc.addupdate_scatter(hist, [b1, lanes], one16)
                smax = jnp.maximum(smax, s)
            return smax
         smax = lax.fori_loop(0, _T // 4, pa, imin, unroll=2)

        ns2 = jax.named_scope("scan1")
        ns2.__enter__()
        startbin = jnp.max(lax.shift_right_arithmetic(smax, sh21) + 1024)

        def sc_cond(st):
            c, _bin, _piv, _above = st
            return jnp.any(c < kvec)

        def sc_body(st):
            c, bin_, piv, above = st
            row = hist[bin_]
            newc = c + row
            take = jnp.logical_and(c < kvec, newc >= kvec)
            piv = jnp.where(take, bin_, piv)
            above = jnp.where(take, c, above)
            return (newc, bin_ - 1, piv, above)
        _, _, piv1, above1 = lax.while_loop(
            sc_cond, sc_body, (zero16, startbin, zero16, zero16))
        rem = kvec - above1
        ns2.__exit__(None, None, None)

        with ns("passB"):
         def pb(i, cnt):
            for c in range(4):
                s = _skey(xblk[i * 4 + c])
                b1 = lax.shift_right_arithmetic(s, sh21) + 1024
                m = jnp.logical_and(b1 >= piv1, cnt < _CAP)
                plsc.store_scatter(buf, [cnt, lanes], s, mask=m)
                cnt = cnt + jnp.where(m, 1, 0)
            return cnt
         cnt = lax.fori_loop(0, _T // 4, pb, zero16, unroll=2)
        maxcnt = jnp.max(cnt)

        ns3 = jax.named_scope("phaseC")
        ns3.__enter__()
        prefix = piv1
        for sh in (14, 7, 0):
            def zh2(i, _):
                hist2[i] = zero16
                return 0
            lax.fori_loop(0, _NB2, zh2, 0, unroll=4)

            def hb(t2, _):
                s = buf[t2]
                u = jnp.bitwise_xor(s, imin)
                valid = cnt > t2
                inplay = jnp.logical_and(valid, _shr(u, sh + 7) == prefix)
                b2 = jnp.bitwise_and(_shr(u, sh), _NB2 - 1)
                plsc.addupdate_scatter(hist2, [b2, lanes], one16, mask=inplay)
                return 0
            lax.fori_loop(0, maxcnt, hb, 0)

            def s2(i, st):
                c, pf, above = st
                sb = _NB2 - 1 - i
                row = hist2[sb]
                newc = c + row
                take = jnp.logical_and(c < rem, newc >= rem)
                pf = jnp.where(take, sb, pf)
                above = jnp.where(take, c, above)
                return (newc, pf, above)
            _, pivr, above_r = lax.fori_loop(0, _NB2, s2, (zero16, zero16, zero16))
            prefix = prefix * _NB2 + pivr
            rem = rem - above_r

        ns3.__exit__(None, None, None)
        thr_s = jnp.bitwise_xor(prefix, imin)

        with ns("passD"):
         def pd(t2, st):
            ocnt, eqc = st
            s = buf[t2]
            valid = cnt > t2
            gt = s > thr_s
            eq = jnp.logical_and(valid, s == thr_s)
            eqok = jnp.logical_and(eq, eqc < rem)
            sel = jnp.logical_and(valid, jnp.logical_or(gt, eqok))
            plsc.store_scatter(obuf_i, [ocnt, lanes], s, mask=sel)
            ocnt = ocnt + jnp.where(sel, 1, 0)
            eqc = eqc + jnp.where(eq, 1, 0)
            return (ocnt, eqc)
         lax.fori_loop(0, maxcnt, pd, (zero16, zero16))

        def cv(j, _):
            sv = obuf_i[j]
            bb = jnp.where(sv < 0,
                           jnp.bitwise_not(jnp.bitwise_xor(sv, imin)), sv)
            obuf_f[j] = lax.bitcast_convert_type(bb, jnp.float32)
            return 0
        lax.fori_loop(0, _K, cv, 0, unroll=4)

        pltpu.sync_copy(obuf_f, o_hbm.at[b, :, pl.ds(dg * 16, 16)])
        return 0

    lax.fori_loop(0, _GRP_PER_W, group_body, 0)


def _sc_call(xg):
    mesh = plsc.VectorSubcoreMesh(core_axis_name="c", subcore_axis_name="s")
    f = pl.kernel(
        _sc_body,
        out_type=jax.ShapeDtypeStruct((4, _K, 1024), jnp.float32),
        mesh=mesh,
        compiler_params=pltpu.CompilerParams(
            needs_layout_passes=False, use_tc_tiling_on_sc=False),
        scratch_types=[
            pltpu.VMEM((_T, 16), jnp.float32),
            pltpu.VMEM((_NB1, 16), jnp.int32),
            pltpu.VMEM((_NB2, 16), jnp.int32),
            pltpu.VMEM((_CAP, 16), jnp.int32),
            pltpu.VMEM((_K, 16), jnp.int32),
            pltpu.VMEM((_K, 16), jnp.float32),
        ],
    )
    return f(xg)


@jax.jit
def kernel(inputs):
    return _sc_call(inputs)


# speedup vs baseline: 2.3237x; 2.3237x over previous
import functools

import jax
import jax.numpy as jnp
from jax import lax
from jax.experimental import pallas as pl
from jax.experimental.pallas import tpu as pltpu
from jax.experimental.pallas import tpu_sc as plsc

_K = 64
_IMIN = -(2 ** 31)
_CAP = 512
_NB1 = 2048
_NBC = 128
_NB2 = 128
_T = 4096
_NGRP = 256
_GRP_PER_W = 8


def _shr(x, n):
    return lax.shift_right_logical(x, lax.full_like(x, n))


def _skey(x):
    b = lax.bitcast_convert_type(x, jnp.int32)
    m = lax.shift_right_arithmetic(b, lax.full_like(b, 31))
    return jnp.bitwise_xor(b, lax.shift_right_logical(m, lax.full_like(m, 1)))


def _sc_body(x_hbm, o_hbm, xblk, hist, hist2, buf, obuf_i, obuf_f):
    wid = lax.axis_index("s") * 2 + lax.axis_index("c")
    lanes = lax.iota(jnp.int32, 16)
    imin = jnp.full((16,), _IMIN, jnp.int32)
    sh21 = jnp.full((16,), 21, jnp.int32)
    one16 = jnp.ones((16,), jnp.int32)
    zero16 = jnp.zeros((16,), jnp.int32)
    kvec = jnp.full((16,), _K, jnp.int32)

    def group_body(gi, _carry):
        g = wid * _GRP_PER_W + gi
        b = g // 64
        dg = g % 64
        pltpu.sync_copy(x_hbm.at[b, :, pl.ds(dg * 16, 16)], xblk)

        def zh(i, _):
            for c in range(8):
                hist[i * 8 + c] = zero16
            return 0
        lax.fori_loop(0, _NB1 // 8, zh, 0, unroll=2)

        def pa(i, st):
            smax = st
            ss = [_skey(xblk[i * 8 + c]) for c in range(8)]
            bs = [lax.shift_right_arithmetic(s, sh21) + 1024 for s in ss]
            m01 = jnp.maximum(ss[0], ss[1])
            m23 = jnp.maximum(ss[2], ss[3])
            m45 = jnp.maximum(ss[4], ss[5])
            m67 = jnp.maximum(ss[6], ss[7])
            m03 = jnp.maximum(m01, m23)
            m47 = jnp.maximum(m45, m67)
            smax = jnp.maximum(smax, jnp.maximum(m03, m47))
            for c in range(8):
                plsc.addupdate_scatter(hist, [bs[c], lanes], one16)
            return smax
        smax = lax.fori_loop(0, _T // 8, pa, imin)

        startbin = jnp.max(lax.shift_right_arithmetic(smax, sh21) + 1024)

        def sc_cond(st):
            c, _bin, _piv, _above = st
            return jnp.any(c < kvec)

        def sc_body(st):
            c, bin_, piv, above = st
            row = hist[bin_]
            newc = c + row
            take = jnp.logical_and(c < kvec, newc >= kvec)
            piv = jnp.where(take, bin_, piv)
            above = jnp.where(take, c, above)
            return (newc, bin_ - 1, piv, above)
        _, _, piv1, above1 = lax.while_loop(
            sc_cond, sc_body, (zero16, startbin, zero16, zero16))
        rem = kvec - above1

        def pb(i, cnt):
            ss = [_skey(xblk[i * 8 + c]) for c in range(8)]
            mi = [
                ((lax.shift_right_arithmetic(s, sh21) + 1024) >= piv1
                 ).astype(jnp.int32)
                for s in ss
            ]
            pos = []
            run = cnt
            for c in range(8):
                pos.append(run)
                run = run + mi[c]
            for c in range(8):
                m = jnp.logical_and(mi[c] > 0, pos[c] < _CAP)
                plsc.store_scatter(buf, [pos[c], lanes], ss[c], mask=m)
            return jnp.minimum(run, jnp.full((16,), _CAP, jnp.int32))
        cnt = lax.fori_loop(0, _T // 8, pb, zero16)
        maxcnt = jnp.max(cnt)

        prefix = piv1
        for sh in (14, 7, 0):
            def zh2(i, _):
                hist2[i] = zero16
                return 0
            lax.fori_loop(0, _NB2, zh2, 0, unroll=4)

            def hb(t2, _):
                s = buf[t2]
                u = jnp.bitwise_xor(s, imin)
                valid = cnt > t2
                inplay = jnp.logical_and(valid, _shr(u, sh + 7) == prefix)
                b2 = jnp.bitwise_and(_shr(u, sh), _NB2 - 1)
                plsc.addupdate_scatter(hist2, [b2, lanes], one16, mask=inplay)
                return 0
            lax.fori_loop(0, maxcnt, hb, 0)

            def s2(i, st):
                c, pf, above = st
                sb = _NB2 - 1 - i
                row = hist2[sb]
                newc = c + row
                take = jnp.logical_and(c < rem, newc >= rem)
                pf = jnp.where(take, sb, pf)
                above = jnp.where(take, c, above)
                return (newc, pf, above)
            _, pivr, above_r = lax.fori_loop(0, _NB2, s2, (zero16, zero16, zero16))
            prefix = prefix * _NB2 + pivr
            rem = rem - above_r

        thr_s = jnp.bitwise_xor(prefix, imin)

        def pd(t2, st):
            ocnt, eqc = st
            s = buf[t2]
            valid = cnt > t2
            gt = s > thr_s
            eq = jnp.logical_and(valid, s == thr_s)
            eqok = jnp.logical_and(eq, eqc < rem)
            sel = jnp.logical_and(valid, jnp.logical_or(gt, eqok))
            plsc.store_scatter(obuf_i, [ocnt, lanes], s, mask=sel)
            ocnt = ocnt + jnp.where(sel, 1, 0)
            eqc = eqc + jnp.where(eq, 1, 0)
            return (ocnt, eqc)
        lax.fori_loop(0, maxcnt, pd, (zero16, zero16))

        def cv(j, _):
            sv = obuf_i[j]
            bb = jnp.where(sv < 0,
                           jnp.bitwise_not(jnp.bitwise_xor(sv, imin)), sv)
            obuf_f[j] = lax.bitcast_convert_type(bb, jnp.float32)
            return 0
        lax.fori_loop(0, _K, cv, 0, unroll=4)

        pltpu.sync_copy(obuf_f, o_hbm.at[b, :, pl.ds(dg * 16, 16)])
        return 0

    lax.fori_loop(0, _GRP_PER_W, group_body, 0)


def _sc_call(xg):
    mesh = plsc.VectorSubcoreMesh(core_axis_name="c", subcore_axis_name="s")
    f = pl.kernel(
        _sc_body,
        out_type=jax.ShapeDtypeStruct((4, _K, 1024), jnp.float32),
        mesh=mesh,
        compiler_params=pltpu.CompilerParams(
            needs_layout_passes=False, use_tc_tiling_on_sc=False),
        scratch_types=[
            pltpu.VMEM((_T, 16), jnp.float32),
            pltpu.VMEM((_NB1, 16), jnp.int32),
            pltpu.VMEM((_NB2, 16), jnp.int32),
            pltpu.VMEM((_CAP, 16), jnp.int32),
            pltpu.VMEM((_K, 16), jnp.int32),
            pltpu.VMEM((_K, 16), jnp.float32),
        ],
    )
    return f(xg)


@jax.jit
def kernel(inputs):
    return _sc_call(inputs)


# speedup vs baseline: 2.6100x; 1.1232x over previous
import functools

import jax
import jax.numpy as jnp
from jax import lax
from jax.experimental import pallas as pl
from jax.experimental.pallas import tpu as pltpu
from jax.experimental.pallas import tpu_sc as plsc

_K = 64
_IMIN = -(2 ** 31)
_CAP = 512
_NB1 = 2048
_NBC = 128
_NB2 = 128
_T = 4096
_NGRP = 256
_GRP_PER_W = 8


def _shr(x, n):
    return lax.shift_right_logical(x, lax.full_like(x, n))


def _skey(x):
    b = lax.bitcast_convert_type(x, jnp.int32)
    m = lax.shift_right_arithmetic(b, lax.full_like(b, 31))
    return jnp.bitwise_xor(b, lax.shift_right_logical(m, lax.full_like(m, 1)))


def _sc_body(x_hbm, o_hbm, xblk, hist, hist2, buf, obuf_i, obuf_f, sem1, sem2):
    wid = lax.axis_index("s") * 2 + lax.axis_index("c")
    lanes = lax.iota(jnp.int32, 16)
    imin = jnp.full((16,), _IMIN, jnp.int32)
    sh21 = jnp.full((16,), 21, jnp.int32)
    one16 = jnp.ones((16,), jnp.int32)
    zero16 = jnp.zeros((16,), jnp.int32)
    kvec = jnp.full((16,), _K, jnp.int32)

    half = _T // 2

    def _fire(g):
        b = g // 64
        dg = g % 64
        pltpu.async_copy(
            x_hbm.at[b, pl.ds(0, half), pl.ds(dg * 16, 16)],
            xblk.at[pl.ds(0, half)], sem1)
        pltpu.async_copy(
            x_hbm.at[b, pl.ds(half, half), pl.ds(dg * 16, 16)],
            xblk.at[pl.ds(half, half)], sem2)

    def _drain(g):
        b = g // 64
        dg = g % 64
        pltpu.make_async_copy(
            x_hbm.at[b, pl.ds(0, half), pl.ds(dg * 16, 16)],
            xblk.at[pl.ds(0, half)], sem1).wait()
        pltpu.make_async_copy(
            x_hbm.at[b, pl.ds(half, half), pl.ds(dg * 16, 16)],
            xblk.at[pl.ds(half, half)], sem2).wait()

    _fire(wid * _GRP_PER_W)

    def group_body(gi, _carry):
        g = wid * _GRP_PER_W + gi
        b = g // 64
        dg = g % 64
        _drain(g)

        def zh(i, _):
            for c in range(8):
                hist[i * 8 + c] = zero16
            return 0
        lax.fori_loop(0, _NB1 // 8, zh, 0, unroll=2)

        def pa(i, st):
            smax = st
            ss = [_skey(xblk[i * 8 + c]) for c in range(8)]
            bs = [lax.shift_right_arithmetic(s, sh21) + 1024 for s in ss]
            m01 = jnp.maximum(ss[0], ss[1])
            m23 = jnp.maximum(ss[2], ss[3])
            m45 = jnp.maximum(ss[4], ss[5])
            m67 = jnp.maximum(ss[6], ss[7])
            m03 = jnp.maximum(m01, m23)
            m47 = jnp.maximum(m45, m67)
            smax = jnp.maximum(smax, jnp.maximum(m03, m47))
            for c in range(8):
                plsc.addupdate_scatter(hist, [bs[c], lanes], one16)
            return smax
        smax = lax.fori_loop(0, _T // 8, pa, imin)

        startbin = jnp.max(lax.shift_right_arithmetic(smax, sh21) + 1024)

        def sc_cond(st):
            c, _bin, _piv, _above = st
            return jnp.any(c < kvec)

        def sc_body(st):
            c, bin_, piv, above = st
            row = hist[bin_]
            newc = c + row
            take = jnp.logical_and(c < kvec, newc >= kvec)
            piv = jnp.where(take, bin_, piv)
            above = jnp.where(take, c, above)
            return (newc, bin_ - 1, piv, above)
        _, _, piv1, above1 = lax.while_loop(
            sc_cond, sc_body, (zero16, startbin, zero16, zero16))
        rem = kvec - above1

        def pb(i, cnt):
            ss = [_skey(xblk[i * 8 + c]) for c in range(8)]
            mi = [
                ((lax.shift_right_arithmetic(s, sh21) + 1024) >= piv1
                 ).astype(jnp.int32)
                for s in ss
            ]
            pos = []
            run = cnt
            for c in range(8):
                pos.append(run)
                run = run + mi[c]
            for c in range(8):
                m = jnp.logical_and(mi[c] > 0, pos[c] < _CAP)
                plsc.store_scatter(buf, [pos[c], lanes], ss[c], mask=m)
            return jnp.minimum(run, jnp.full((16,), _CAP, jnp.int32))
        cnt = lax.fori_loop(0, _T // 8, pb, zero16)
        gnext = jnp.minimum(g + 1, wid * _GRP_PER_W + _GRP_PER_W - 1)
        _fire(gnext)
        maxcnt = jnp.max(cnt)

        prefix = piv1
        for sh in (14, 7, 0):
            def zh2(i, _):
                hist2[i] = zero16
                return 0
            lax.fori_loop(0, _NB2, zh2, 0, unroll=4)

            def hb(t2, _):
                s = buf[t2]
                u = jnp.bitwise_xor(s, imin)
                valid = cnt > t2
                inplay = jnp.logical_and(valid, _shr(u, sh + 7) == prefix)
                b2 = jnp.bitwise_and(_shr(u, sh), _NB2 - 1)
                plsc.addupdate_scatter(hist2, [b2, lanes], one16, mask=inplay)
                return 0
            lax.fori_loop(0, maxcnt, hb, 0)

            def s2(i, st):
                c, pf, above = st
                sb = _NB2 - 1 - i
                row = hist2[sb]
                newc = c + row
                take = jnp.logical_and(c < rem, newc >= rem)
                pf = jnp.where(take, sb, pf)
                above = jnp.where(take, c, above)
                return (newc, pf, above)
            _, pivr, above_r = lax.fori_loop(0, _NB2, s2, (zero16, zero16, zero16))
            prefix = prefix * _NB2 + pivr
            rem = rem - above_r

        thr_s = jnp.bitwise_xor(prefix, imin)

        def pd(t2, st):
            ocnt, eqc = st
            s = buf[t2]
            valid = cnt > t2
            gt = s > thr_s
            eq = jnp.logical_and(valid, s == thr_s)
            eqok = jnp.logical_and(eq, eqc < rem)
            sel = jnp.logical_and(valid, jnp.logical_or(gt, eqok))
            plsc.store_scatter(obuf_i, [ocnt, lanes], s, mask=sel)
            ocnt = ocnt + jnp.where(sel, 1, 0)
            eqc = eqc + jnp.where(eq, 1, 0)
            return (ocnt, eqc)
        lax.fori_loop(0, maxcnt, pd, (zero16, zero16))

        def cv(j, _):
            sv = obuf_i[j]
            bb = jnp.where(sv < 0,
                           jnp.bitwise_not(jnp.bitwise_xor(sv, imin)), sv)
            obuf_f[j] = lax.bitcast_convert_type(bb, jnp.float32)
            return 0
        lax.fori_loop(0, _K, cv, 0, unroll=4)

        pltpu.sync_copy(obuf_f, o_hbm.at[b, :, pl.ds(dg * 16, 16)])
        return 0

    lax.fori_loop(0, _GRP_PER_W, group_body, 0)
    _drain(wid * _GRP_PER_W)


def _sc_call(xg):
    mesh = plsc.VectorSubcoreMesh(core_axis_name="c", subcore_axis_name="s")
    f = pl.kernel(
        _sc_body,
        out_type=jax.ShapeDtypeStruct((4, _K, 1024), jnp.float32),
        mesh=mesh,
        compiler_params=pltpu.CompilerParams(
            needs_layout_passes=False, use_tc_tiling_on_sc=False),
        scratch_types=[
            pltpu.VMEM((_T, 16), jnp.float32),
            pltpu.VMEM((_NB1, 16), jnp.int32),
            pltpu.VMEM((_NB2, 16), jnp.int32),
            pltpu.VMEM((_CAP, 16), jnp.int32),
            pltpu.VMEM((_K, 16), jnp.int32),
            pltpu.VMEM((_K, 16), jnp.float32),
            pltpu.SemaphoreType.DMA,
            pltpu.SemaphoreType.DMA,
        ],
    )
    return f(xg)


@jax.jit
def kernel(inputs):
    return _sc_call(inputs)
